# baseline (speedup 1.0000x reference)
import functools

import jax
import jax.numpy as jnp
from jax import lax
from jax.experimental import pallas as pl
from jax.experimental.pallas import tpu as pltpu
from jax.experimental.pallas import tpu_sc as plsc

_VOCAB = 1000000
_D = 32
_BATCH = 16384
_HIST = 50

_NC = 2
_NS = 16
_NW = _NC * _NS

_BLK = 128
_BPW = _BATCH // (_NW * _BLK)
_RING = 2 * _BPW


_TBLK = 16384


def _tr_body(x_ref, y_ref):
    t = x_ref[...].T.reshape(_TBLK // 4, 4, _D)
    y_ref[...] = jnp.concatenate([t[:, c, :] for c in range(4)], axis=1)


_transpose = pl.pallas_call(
    _tr_body,
    grid=((_VOCAB + _TBLK - 1) // _TBLK,),
    in_specs=[pl.BlockSpec((_D, _TBLK), lambda i: (0, i))],
    out_specs=pl.BlockSpec((_TBLK // 4, 128), lambda i: (i, 0)),
    out_shape=jax.ShapeDtypeStruct((_VOCAB // 4, 128), jnp.float32),
)


def _make_gather():
    mesh = plsc.VectorSubcoreMesh(core_axis_name="c", subcore_axis_name="s")

    @functools.partial(
        pl.kernel,
        mesh=mesh,
        out_type=jax.ShapeDtypeStruct(
            (_HIST, _D // 8, _BATCH // _BLK, 8, _BLK), jnp.float32
        ),
        compiler_params=pltpu.CompilerParams(
            use_tc_tiling_on_sc=False, needs_layout_passes=False
        ),
        scratch_types=(
            [pltpu.VMEM((_HIST, _BPW, _BLK), jnp.int32)]
            + [pltpu.VMEM((_BLK, _D), jnp.float32) for _ in range(_RING)]
            + [pltpu.VMEM((_D // 8, 8, _BLK + 1), jnp.float32) for _ in range(_RING)]
            + [pltpu.SemaphoreType.DMA for _ in range(2 * _RING)]
        ),
    )
    def gather_kernel(idx_hbm, table_hbm, out_hbm, idx_all, *bufs):
        rows = bufs[:_RING]
        tbuf = bufs[_RING:2 * _RING]
        sem_g = bufs[2 * _RING:3 * _RING]
        sem_w = bufs[3 * _RING:4 * _RING]
        wid = lax.axis_index("s") * _NC + lax.axis_index("c")

        pltpu.sync_copy(idx_hbm.at[:, pl.ds(wid * _BPW, _BPW), :], idx_all)

        d16 = lax.iota(jnp.int32, 16)
        ds_lo, s_lo = d16 // 8, d16 % 8
        ds_hi = ds_lo + 2

        for b in range(_RING):
            pltpu.async_copy(
                table_hbm.at[idx_all.at[b // _BPW, b % _BPW]], rows[b], sem_g[b]
            )

        def body(i0, carry):
            for b in range(_RING):
                i = 2 * i0 + b // _BPW
                j = b % _BPW
                pltpu.make_async_copy(
                    table_hbm.at[pl.ds(0, _BLK)], rows[b], sem_g[b]
                ).wait()
                @pl.when(i0 > 0)
                def _():
                    pltpu.make_async_copy(
                        tbuf[b].at[:, :, pl.ds(0, _BLK)],
                        out_hbm.at[0, :, 0],
                        sem_w[b],
                    ).wait()
                for l in range(_BLK):
                    lvec = jnp.full((16,), l, jnp.int32)
                    lo = rows[b][l, pl.ds(0, 16)]
                    hi = rows[b][l, pl.ds(16, 16)]
                    plsc.store_scatter(tbuf[b], [ds_lo, s_lo, lvec], lo)
                    plsc.store_scatter(tbuf[b], [ds_hi, s_lo, lvec], hi)
                @pl.when(i < _HIST - 2)
                def _():
                    pltpu.async_copy(
                        table_hbm.at[idx_all.at[i + 2, j]], rows[b], sem_g[b]
                    )
                pltpu.async_copy(
                    tbuf[b].at[:, :, pl.ds(0, _BLK)],
                    out_hbm.at[i, :, wid * _BPW + j],
                    sem_w[b],
                )
            return carry

        lax.fori_loop(0, _HIST // 2, body, 0)
        for j in range(_RING):
            pltpu.make_async_copy(
                tbuf[j].at[:, :, pl.ds(0, _BLK)], out_hbm.at[0, :, 0], sem_w[j]
            ).wait()

    return gather_kernel


_gather = _make_gather()


def kernel(season_ID, table):
    idx = season_ID.astype(jnp.int32).T.reshape(_HIST, _BATCH // _BLK, _BLK)
    table_rm = _transpose(table.T).reshape(_VOCAB, _D)
    out_p = _gather(idx, table_rm)
    return out_p.transpose(2, 4, 0, 1, 3).reshape(_BATCH, _HIST, _D)


# speedup vs baseline: 1.0518x; 1.0518x over previous
import functools

import jax
import jax.numpy as jnp
from jax import lax
from jax.experimental import pallas as pl
from jax.experimental.pallas import tpu as pltpu
from jax.experimental.pallas import tpu_sc as plsc

_VOCAB = 1000000
_D = 32
_BATCH = 16384
_HIST = 50

_NC = 2
_NS = 16
_NW = _NC * _NS

_BLK = 128
_BPW = _BATCH // (_NW * _BLK)
_RING = _BPW


_TBLK = 16384


def _tr_body(x_ref, y_ref):
    t = x_ref[...].T.reshape(_TBLK // 4, 4, _D)
    y_ref[...] = jnp.concatenate([t[:, c, :] for c in range(4)], axis=1)


_transpose = pl.pallas_call(
    _tr_body,
    grid=((_VOCAB + _TBLK - 1) // _TBLK,),
    in_specs=[pl.BlockSpec((_D, _TBLK), lambda i: (0, i))],
    out_specs=pl.BlockSpec((_TBLK // 4, 128), lambda i: (i, 0)),
    out_shape=jax.ShapeDtypeStruct((_VOCAB // 4, 128), jnp.float32),
)


def _make_gather():
    mesh = plsc.VectorSubcoreMesh(core_axis_name="c", subcore_axis_name="s")

    @functools.partial(
        pl.kernel,
        mesh=mesh,
        out_type=jax.ShapeDtypeStruct(
            (_HIST, _D // 8, _BATCH // _BLK, 8, _BLK), jnp.float32
        ),
        compiler_params=pltpu.CompilerParams(
            use_tc_tiling_on_sc=False, needs_layout_passes=False
        ),
        scratch_types=(
            [pltpu.VMEM((_HIST, _BPW, _BLK), jnp.int32)]
            + [pltpu.VMEM((_BLK, _D), jnp.float32) for _ in range(_RING)]
            + [pltpu.VMEM((_D // 8, 8, _BLK + 1), jnp.float32) for _ in range(_RING)]
            + [pltpu.SemaphoreType.DMA for _ in range(2 * _RING)]
        ),
    )
    def gather_kernel(idx_hbm, table_hbm, out_hbm, idx_all, *bufs):
        rows = bufs[:_RING]
        tbuf = bufs[_RING:2 * _RING]
        sem_g = bufs[2 * _RING:3 * _RING]
        sem_w = bufs[3 * _RING:4 * _RING]
        wid = lax.axis_index("s") * _NC + lax.axis_index("c")

        pltpu.sync_copy(idx_hbm.at[:, pl.ds(wid * _BPW, _BPW), :], idx_all)

        d16 = lax.iota(jnp.int32, 16)
        ds_lo, s_lo = d16 // 8, d16 % 8
        ds_hi = ds_lo + 2

        for b in range(_RING):
            pltpu.async_copy(table_hbm.at[idx_all.at[0, b]], rows[b], sem_g[b])

        def body(i, carry):
            for b in range(_RING):
                pltpu.make_async_copy(
                    table_hbm.at[pl.ds(0, _BLK)], rows[b], sem_g[b]
                ).wait()
                @pl.when(i > 0)
                def _():
                    pltpu.make_async_copy(
                        tbuf[b].at[:, :, pl.ds(0, _BLK)],
                        out_hbm.at[0, :, 0],
                        sem_w[b],
                    ).wait()
                for l in range(_BLK):
                    lvec = jnp.full((16,), l, jnp.int32)
                    lo = rows[b][l, pl.ds(0, 16)]
                    hi = rows[b][l, pl.ds(16, 16)]
                    plsc.store_scatter(tbuf[b], [ds_lo, s_lo, lvec], lo)
                    plsc.store_scatter(tbuf[b], [ds_hi, s_lo, lvec], hi)
                @pl.when(i < _HIST - 1)
                def _():
                    pltpu.async_copy(
                        table_hbm.at[idx_all.at[i + 1, b]], rows[b], sem_g[b]
                    )
                pltpu.async_copy(
                    tbuf[b].at[:, :, pl.ds(0, _BLK)],
                    out_hbm.at[i, :, wid * _BPW + b],
                    sem_w[b],
                )
            return carry

        lax.fori_loop(0, _HIST, body, 0)
        for j in range(_RING):
            pltpu.make_async_copy(
                tbuf[j].at[:, :, pl.ds(0, _BLK)], out_hbm.at[0, :, 0], sem_w[j]
            ).wait()

    return gather_kernel


_gather = _make_gather()


def kernel(season_ID, table):
    idx = season_ID.astype(jnp.int32).T.reshape(_HIST, _BATCH // _BLK, _BLK)
    table_rm = _transpose(table.T).reshape(_VOCAB, _D)
    out_p = _gather(idx, table_rm)
    return out_p.transpose(2, 4, 0, 1, 3).reshape(_BATCH, _HIST, _D)
